# baseline (speedup 1.0000x reference)
import functools

import jax
import jax.numpy as jnp
from jax import lax
from jax.experimental import pallas as pl
from jax.experimental.pallas import tpu as pltpu
from jax.experimental.pallas import tpu_sc as plsc

_B, _S, _D = 1024, 200, 128
_NC, _NS, _L = 2, 16, 16
_NW = _NC * _NS
_BPW = _B // _NW
_VPR = _D // _L


def _body(x_hbm, tab_hbm, out_hbm, tab_v, b0, b1, in0, in1, ot0, ot1):
    wid = lax.axis_index("s") * _NC + lax.axis_index("c")
    base = wid * _BPW

    pltpu.sync_copy(tab_hbm, tab_v)

    bufs = (b0, b1)
    insems = (in0, in1)
    outsems = (ot0, ot1)

    def compute(buf):
        def row(i, carry):
            for j in range(_VPR):
                sl = pl.ds(j * _L, _L)
                plsc.addupdate(buf.at[i, sl], tab_v[i, sl])
            return carry

        lax.fori_loop(0, _S, row, 0, unroll=2)

    pltpu.async_copy(x_hbm.at[base], b0, in0)
    pltpu.async_copy(x_hbm.at[base + 1], b1, in1)

    def outer(g2, carry):
        for p in range(2):
            g = g2 * 2 + p
            b = base + g
            pltpu.make_async_copy(x_hbm.at[b], bufs[p], insems[p]).wait()
            compute(bufs[p])
            pltpu.async_copy(bufs[p], out_hbm.at[b], outsems[p])
            pltpu.make_async_copy(bufs[p], out_hbm.at[b], outsems[p]).wait()
            pltpu.async_copy(x_hbm.at[b + 2], bufs[p], insems[p])
        return carry

    lax.fori_loop(0, _BPW // 2 - 1, outer, 0)

    for p in range(2):
        b = base + _BPW - 2 + p
        pltpu.make_async_copy(x_hbm.at[b], bufs[p], insems[p]).wait()
        compute(bufs[p])
        pltpu.async_copy(bufs[p], out_hbm.at[b], outsems[p])
    for p in range(2):
        b = base + _BPW - 2 + p
        pltpu.make_async_copy(bufs[p], out_hbm.at[b], outsems[p]).wait()


_sc_add = functools.partial(
    pl.kernel,
    out_type=jax.ShapeDtypeStruct((_B, _S, _D), jnp.float32),
    mesh=plsc.VectorSubcoreMesh(core_axis_name="c", subcore_axis_name="s"),
    scratch_types=[
        pltpu.VMEM((_S, _D), jnp.float32),
        pltpu.VMEM((_S, _D), jnp.float32),
        pltpu.VMEM((_S, _D), jnp.float32),
        pltpu.SemaphoreType.DMA,
        pltpu.SemaphoreType.DMA,
        pltpu.SemaphoreType.DMA,
        pltpu.SemaphoreType.DMA,
    ],
)(_body)


@jax.jit
def kernel(x, pos_emb_weight):
    return _sc_add(x, pos_emb_weight)


# speedup vs baseline: 1.1459x; 1.1459x over previous
import functools

import jax
import jax.numpy as jnp
from jax import lax
from jax.experimental import pallas as pl
from jax.experimental.pallas import tpu as pltpu
from jax.experimental.pallas import tpu_sc as plsc

_B, _S, _D = 1024, 200, 128
_NC, _NS, _L = 2, 16, 16
_NW = _NC * _NS
_BPW = _B // _NW
_VPR = _D // _L


_RING = 4


def _body(x_hbm, tab_hbm, out_hbm, tab_v, b0, b1, b2, b3,
          i0, i1, i2, i3, o0, o1, o2, o3):
    bufs = (b0, b1, b2, b3)
    insems = (i0, i1, i2, i3)
    outsems = (o0, o1, o2, o3)
    wid = lax.axis_index("s") * _NC + lax.axis_index("c")
    base = wid * _BPW

    pltpu.sync_copy(tab_hbm, tab_v)

    def compute(buf):
        def row(i, carry):
            for j in range(_VPR):
                sl = pl.ds(j * _L, _L)
                plsc.addupdate(buf.at[i, sl], tab_v[i, sl])
            return carry

        lax.fori_loop(0, _S, row, 0, unroll=2)

    for p in range(_RING):
        pltpu.async_copy(x_hbm.at[base + p], bufs[p], insems[p])

    def outer(g4, carry):
        for p in range(_RING):
            g = g4 * _RING + p
            b = base + g
            pltpu.make_async_copy(x_hbm.at[b], bufs[p], insems[p]).wait()
            compute(bufs[p])
            pltpu.async_copy(bufs[p], out_hbm.at[b], outsems[p])
            h = g + 2
            q = (p + 2) % _RING

            @pl.when(jnp.logical_and(h >= _RING, h < _BPW))
            def _():
                pltpu.make_async_copy(
                    bufs[q], out_hbm.at[base + h - _RING], outsems[q]
                ).wait()
                pltpu.async_copy(x_hbm.at[base + h], bufs[q], insems[q])

        return carry

    lax.fori_loop(0, _BPW // _RING, outer, 0)

    for p in range(_RING):
        b = base + _BPW - _RING + p
        pltpu.make_async_copy(bufs[p], out_hbm.at[b], outsems[p]).wait()


_sc_add = functools.partial(
    pl.kernel,
    out_type=jax.ShapeDtypeStruct((_B, _S, _D), jnp.float32),
    mesh=plsc.VectorSubcoreMesh(core_axis_name="c", subcore_axis_name="s"),
    scratch_types=(
        [pltpu.VMEM((_S, _D), jnp.float32)] * (1 + _RING)
        + [pltpu.SemaphoreType.DMA] * (2 * _RING)
    ),
)(_body)


@jax.jit
def kernel(x, pos_emb_weight):
    return _sc_add(x, pos_emb_weight)
